# Initial kernel scaffold; baseline (speedup 1.0000x reference)
import jax
import jax.numpy as jnp
from jax.experimental import pallas as pl


def kernel(x, edge_index, params):
    raise NotImplementedError("write your pallas kernel here")


# speedup vs baseline: 31.6055x; 31.6055x over previous
import functools

import jax
import jax.numpy as jnp
from jax import lax
from jax.experimental import pallas as pl
from jax.experimental.pallas import tpu as pltpu
from jax.experimental.pallas import tpu_sc as plsc

_N = 10000
_E = 320000
_HID = 128
_H = 8
_C = 16
_BNECK = 32
_WIDE = 256

_NCORE = 2
_NTILE = 16
_NW = _NCORE * _NTILE
_K = 32
_NCHUNK = 312
_EPWM = _NCHUNK * _K
_TAIL0 = _NW * _EPWM
_NPAD = 10240
_RPT = _NPAD // _NTILE
_NPW = _NPAD // 8
_RPTW = _NPW // _NTILE
_ZB = 32


def _bn(x, g, b):
    mu = jnp.mean(x, axis=0, keepdims=True)
    var = jnp.mean((x - mu) ** 2, axis=0, keepdims=True)
    return (x - mu) * lax.rsqrt(var + 1e-5) * g + b


def _elu(x):
    return jnp.where(x > 0, x, jnp.exp(x) - 1.0)


def _pre_body(x_ref, w_ref, asd_ref, hs_ref, ad_ref):
    h = jnp.dot(x_ref[...], w_ref[...], preferred_element_type=jnp.float32)
    asd = jnp.dot(h, asd_ref[...], preferred_element_type=jnp.float32)
    z = jnp.zeros((_N, _WIDE - _HID - _H), jnp.float32)
    hs_ref[...] = jnp.concatenate([h, asd[:, :_H], z], axis=1)
    ad_ref[...] = jnp.concatenate(
        [asd[:, _H:], jnp.zeros((_N, _HID - _H), jnp.float32)], axis=1)


def _pre_call(x, w, asd):
    return pl.pallas_call(
        _pre_body,
        out_shape=(
            jax.ShapeDtypeStruct((_N, _WIDE), jnp.float32),
            jax.ShapeDtypeStruct((_N, _HID), jnp.float32),
        ),
    )(x, w, asd)


def _post_body(acc_ref, accw_ref, xin_ref, wres_ref, bexp_ref, bias_ref,
               bng_ref, bnb_ref, dw_ref, db_ref, dg_ref, dbb_ref, uw_ref,
               ub_ref, ug_ref, ubb_ref, xout_ref):
    out_u = acc_ref[0, :_N] + acc_ref[1, :_N]
    wp = accw_ref[0] + accw_ref[1]
    bexp = bexp_ref[...]
    den = jnp.stack(
        [jnp.dot(wp[:, m * 16:(m + 1) * 16], bexp,
                 preferred_element_type=jnp.float32) for m in range(8)],
        axis=1).reshape(_NPAD, _HID)[:_N] + 1e-16
    xin = xin_ref[...]
    g = (out_u / den
         + jnp.dot(xin, wres_ref[...], preferred_element_type=jnp.float32)
         + bias_ref[...])
    x1 = _elu(_bn(g, bng_ref[...], bnb_ref[...]))
    d = _elu(_bn(jnp.dot(x1, dw_ref[...], preferred_element_type=jnp.float32)
                 + db_ref[...], dg_ref[...], dbb_ref[...]))
    u = _elu(_bn(jnp.dot(d, uw_ref[...], preferred_element_type=jnp.float32)
                 + ub_ref[...], ug_ref[...], ubb_ref[...]))
    xout_ref[...] = x1 + u + xin


def _post_call(acc, accw, xin, wres, bexp, bias, bng, bnb, dw, db, dg, dbb,
               uw, ub, ug, ubb):
    return pl.pallas_call(
        _post_body,
        out_shape=jax.ShapeDtypeStruct((_N, _HID), jnp.float32),
    )(acc, accw, xin, wres, bexp, bias, bng, bnb, dw, db, dg, dbb, uw, ub,
      ug, ubb)


def _fin_body(x_ref, o_ref):
    x = x_ref[...]
    mu = jnp.mean(x, axis=0, keepdims=True)
    xc = x - mu
    var = jnp.sum(xc * xc, axis=0, keepdims=True) / (_N - 1)
    x = xc / (jnp.sqrt(var) + 1e-6)
    n = jnp.sqrt(jnp.sum(x * x, axis=1, keepdims=True))
    o_ref[...] = x / jnp.maximum(n, 1e-12)


def _fin_call(x):
    return pl.pallas_call(
        _fin_body,
        out_shape=jax.ShapeDtypeStruct((_N, _HID), jnp.float32),
    )(x)


_SPLAT_DNUMS = lax.GatherDimensionNumbers(
    offset_dims=(), collapsed_slice_dims=(0,), start_index_map=(0,))


def _splat(v16, lane):
    return lax.gather(v16, jnp.full((16, 1), lane, jnp.int32), _SPLAT_DNUMS,
                      (1,), mode=lax.GatherScatterMode.PROMISE_IN_BOUNDS)


def _sc_body(hs_hbm, ad_hbm, src_hbm, dst_hbm, out_hbm, outw_hbm,
             acc_sh, accw_sh, src_v, dst_v, dstp_v, rows_v, adrows_v,
             scaled_v, wp_v, zb_v, sem1, sem2):
    cid = lax.axis_index("c")
    sid = lax.axis_index("s")
    wid = cid * _NTILE + sid
    zeros16 = jnp.zeros((16,), jnp.float32)

    for r in range(_ZB):
        for cs in range(_HID // 16):
            zb_v[r, pl.ds(cs * 16, 16)] = zeros16
    row0 = sid * _RPT
    for i in range(_RPT // _ZB):
        pltpu.sync_copy(zb_v, acc_sh.at[pl.ds(row0 + i * _ZB, _ZB)])
    roww0 = sid * _RPTW
    for i in range(_RPTW // _ZB):
        pltpu.sync_copy(zb_v, accw_sh.at[pl.ds(roww0 + i * _ZB, _ZB)])
    pltpu.sync_copy(zb_v.at[pl.ds(0, _RPTW % _ZB)],
                    accw_sh.at[pl.ds(roww0 + (_RPTW // _ZB) * _ZB,
                                     _RPTW % _ZB)])
    plsc.subcore_barrier()

    def process(base):
        pltpu.sync_copy(src_hbm.at[pl.ds(base, _K)], src_v)
        pltpu.sync_copy(dst_hbm.at[pl.ds(base, _K)], dst_v)
        g1 = pltpu.async_copy(hs_hbm.at[src_v], rows_v, sem1)
        g2 = pltpu.async_copy(ad_hbm.at[dst_v], adrows_v, sem2)

        def grp_idx(g, c):
            j0 = pl.multiple_of(g * 16, 8)
            dv = dst_v[pl.ds(j0, 16)]
            dstp_v[pl.ds(j0, 16)] = lax.shift_right_logical(dv, 3)
            return c

        lax.fori_loop(0, _K // 16, grp_idx, 0)
        g1.wait()
        g2.wait()

        def grp(g, c):
            j0 = pl.multiple_of(g * 16, 8)
            dmod = jnp.bitwise_and(dst_v[pl.ds(j0, 16)], 7)
            for t in range(16):
                j = j0 + t
                av = rows_v[j, pl.ds(_HID, 16)]
                adv = adrows_v[j, pl.ds(0, 16)]
                al = av + adv
                al = jnp.maximum(al, al * 0.2)
                w16 = jnp.exp(al)
                for h in range(_H):
                    scaled_v[j, pl.ds(h * 16, 16)] = (
                        rows_v[j, pl.ds(h * 16, 16)] * _splat(w16, h))
                dspl = _splat(dmod, t)
                for b in range(8):
                    mf = jnp.maximum(
                        1 - jnp.abs(dspl - b), 0).astype(jnp.float32)
                    wp_v[j, pl.ds(b * 16, 16)] = w16 * mf
            return c

        lax.fori_loop(0, _K // 16, grp, 0)
        pltpu.sync_copy(scaled_v, acc_sh.at[dst_v], add=True)
        pltpu.sync_copy(wp_v, accw_sh.at[dstp_v], add=True)

    def chunk(k, carry):
        process(pl.multiple_of(wid * _EPWM + k * _K, 8))
        return carry

    lax.fori_loop(0, _NCHUNK, chunk, 0)

    @pl.when(wid < _NW // 2)
    def _tail():
        process(pl.multiple_of(_TAIL0 + wid * _K, 8))

    plsc.subcore_barrier()
    pltpu.sync_copy(acc_sh.at[pl.ds(row0, _RPT)],
                    out_hbm.at[cid, pl.ds(row0, _RPT)])
    pltpu.sync_copy(accw_sh.at[pl.ds(roww0, _RPTW)],
                    outw_hbm.at[cid, pl.ds(roww0, _RPTW)])


_sc_call = pl.kernel(
    _sc_body,
    out_type=(
        jax.ShapeDtypeStruct((_NCORE, _NPAD, _HID), jnp.float32),
        jax.ShapeDtypeStruct((_NCORE, _NPW, _HID), jnp.float32),
    ),
    mesh=plsc.VectorSubcoreMesh(core_axis_name="c", subcore_axis_name="s"),
    scratch_types=[
        pltpu.VMEM_SHARED((_NPAD, _HID), jnp.float32),
        pltpu.VMEM_SHARED((_NPW, _HID), jnp.float32),
        pltpu.VMEM((_K,), jnp.int32),
        pltpu.VMEM((_K,), jnp.int32),
        pltpu.VMEM((_K,), jnp.int32),
        pltpu.VMEM((_K, _WIDE), jnp.float32),
        pltpu.VMEM((_K, _HID), jnp.float32),
        pltpu.VMEM((_K, _HID), jnp.float32),
        pltpu.VMEM((_K, _HID), jnp.float32),
        pltpu.VMEM((_ZB, _HID), jnp.float32),
        pltpu.SemaphoreType.DMA,
        pltpu.SemaphoreType.DMA,
    ],
)


def kernel(x, edge_index, params):
    src = edge_index[0]
    dst = edge_index[1]
    eye = jnp.eye(_H, dtype=jnp.float32)
    bexp = jnp.concatenate(
        [jnp.kron(eye, jnp.ones((1, _C), jnp.float32)),
         jnp.zeros((_H, _HID), jnp.float32)], axis=0)
    out = x
    for p in params['layers']:
        a_src = (p['att_src'][0][:, :, None] * eye[:, None, :]).reshape(_HID, _H)
        a_dst = (p['att_dst'][0][:, :, None] * eye[:, None, :]).reshape(_HID, _H)
        asd = jnp.concatenate([a_src, a_dst], axis=1)
        hs, ad = _pre_call(out, p['W'], asd)
        acc, accw = _sc_call(hs, ad, src, dst)
        out = _post_call(
            acc, accw, out, p['W_res'], bexp,
            p['bias'].reshape(1, _HID),
            p['bn_g'].reshape(1, _HID), p['bn_b'].reshape(1, _HID),
            p['down_W'], p['down_b'].reshape(1, _BNECK),
            p['down_g'].reshape(1, _BNECK), p['down_bb'].reshape(1, _BNECK),
            p['up_W'], p['up_b'].reshape(1, _HID),
            p['up_g'].reshape(1, _HID), p['up_bb'].reshape(1, _HID))
    return _fin_call(out)
